# baseline (speedup 1.0000x reference)
import jax
import jax.numpy as jnp
from jax import lax
from jax.experimental import pallas as pl
from jax.experimental.pallas import tpu as pltpu
from jax.experimental.pallas import tpu_sc as plsc

VOCAB = 1000
B, T = 1024, 50
N_TOK = B * T
NC, NS, L = 2, 16, 16
NW = NC * NS
VT = VOCAB // 8
N_UNITS = VT * T
UNIT = 8 * 8 * 128
PER_TILE = N_TOK // NW
CHUNK = 64
N_CHUNKS = PER_TILE // CHUNK
GROUPS = CHUNK // L


def _lse_body(table_ref, out_ref):
    t = table_ref[...]
    m = jnp.max(t, axis=1, keepdims=True)
    out_ref[...] = m[:, 0] + jnp.log(jnp.sum(jnp.exp(t - m), axis=1))


def _gather_body(idxT_hbm, table_hbm, out_hbm, idxv, slab_v, outbuf):
    cid = lax.axis_index("c")
    sid = lax.axis_index("s")
    wid = sid * NC + cid
    ustart = wid * N_UNITS // NW
    uend = (wid + 1) * N_UNITS // NW

    pltpu.sync_copy(idxT_hbm, idxv)

    def unit_body(u, carry):
        vt = u // T
        t = u % T

        @pl.when(jnp.logical_or(t == 0, u == ustart))
        def _():
            pltpu.sync_copy(table_hbm.at[pl.ds(0, VOCAB), pl.ds(vt * 8, 8)],
                            slab_v)

        def m_body(m, c2):
            bt = m // 8
            g = m % 8
            idx16 = idxv[t, pl.ds(bt * 128 + g * 16, L)]
            for vs in range(8):
                val = plsc.load_gather(
                    slab_v, [idx16, jnp.full((L,), vs, jnp.int32)])
                outbuf[pl.ds(bt * 1024 + vs * 128 + g * 16, L)] = val
            return c2

        lax.fori_loop(0, 64, m_body, 0)
        pltpu.sync_copy(outbuf, out_hbm.at[pl.ds((t * VT + vt) * UNIT, UNIT)])
        return carry

    lax.fori_loop(ustart, uend, unit_body, 0)


def _loss_body(idx_hbm, tgt_hbm, table1_hbm, lse_hbm, loss_hbm,
               idx_v, tgt_v, fidx_c, tscal_v, lscal_v,
               accv, sums_v, lossv, shared):
    cid = lax.axis_index("c")
    sid = lax.axis_index("s")
    wid = sid * NC + cid
    base = wid * PER_TILE

    pltpu.sync_copy(idx_hbm.at[pl.ds(base, PER_TILE)], idx_v)
    pltpu.sync_copy(tgt_hbm.at[pl.ds(base, PER_TILE)], tgt_v)

    def chunk(c, acc):
        off = c * CHUNK
        for g in range(GROUPS):
            o = off + g * L
            fidx_c[pl.ds(g * L, L)] = idx_v[pl.ds(o, L)] * VOCAB + tgt_v[pl.ds(o, L)]
        pltpu.sync_copy(table1_hbm.at[fidx_c], tscal_v)
        pltpu.sync_copy(lse_hbm.at[idx_v.at[pl.ds(off, CHUNK)]], lscal_v)
        for g in range(GROUPS):
            acc = acc + lscal_v[pl.ds(g * L, L)] - tscal_v[pl.ds(g * L, L)]
        return acc

    acc = lax.fori_loop(0, N_CHUNKS, chunk, jnp.zeros((L,), jnp.float32))
    accv[...] = acc
    pltpu.sync_copy(accv, shared.at[sid])
    plsc.subcore_barrier()

    @pl.when(sid == 0)
    def _():
        pltpu.sync_copy(shared, sums_v)
        tot = sums_v[0]
        for j in range(1, NS):
            tot = tot + sums_v[j]
        lossv[...] = tot * (1.0 / N_TOK)
        pltpu.sync_copy(lossv, loss_hbm.at[cid])


def kernel(idx, targets, table):
    lse = pl.pallas_call(
        _lse_body,
        out_shape=jax.ShapeDtypeStruct((VOCAB,), jnp.float32),
    )(table)

    mesh = plsc.VectorSubcoreMesh(core_axis_name="c", subcore_axis_name="s")
    params = pltpu.CompilerParams(use_tc_tiling_on_sc=False,
                                  needs_layout_passes=False)

    gather = pl.kernel(
        _gather_body,
        out_type=jax.ShapeDtypeStruct((N_TOK * VOCAB,), jnp.float32),
        mesh=mesh,
        compiler_params=params,
        scratch_types=[
            pltpu.VMEM((T, B), jnp.int32),
            pltpu.VMEM((VOCAB, 8), jnp.float32),
            pltpu.VMEM((UNIT,), jnp.float32),
        ],
    )
    flat = gather(jnp.transpose(idx), table)
    o5 = flat.reshape(T, VT, 8, 8, 128)
    logits = o5.transpose(2, 4, 0, 1, 3).reshape(B, T, VOCAB)

    loss_k = pl.kernel(
        _loss_body,
        out_type=jax.ShapeDtypeStruct((NC, L), jnp.float32),
        mesh=mesh,
        compiler_params=params,
        scratch_types=[
            pltpu.VMEM((PER_TILE,), jnp.int32),
            pltpu.VMEM((PER_TILE,), jnp.int32),
            pltpu.VMEM((CHUNK,), jnp.int32),
            pltpu.VMEM((CHUNK,), jnp.float32),
            pltpu.VMEM((CHUNK,), jnp.float32),
            pltpu.VMEM((L,), jnp.float32),
            pltpu.VMEM((NS, L), jnp.float32),
            pltpu.VMEM((L,), jnp.float32),
            pltpu.VMEM_SHARED((NS, L), jnp.float32),
        ],
    )
    loss_parts = loss_k(idx.reshape(N_TOK), targets.reshape(N_TOK),
                        table.reshape(VOCAB * VOCAB), lse)
    loss = jnp.sum(loss_parts)
    return (logits, loss)


# speedup vs baseline: 1.0847x; 1.0847x over previous
import jax
import jax.numpy as jnp
from jax import lax
from jax.experimental import pallas as pl
from jax.experimental.pallas import tpu as pltpu
from jax.experimental.pallas import tpu_sc as plsc

VOCAB = 1000
B, T = 1024, 50
N_TOK = B * T
NC, NS, L = 2, 16, 16
NW = NC * NS
VT = VOCAB // 8
N_UNITS = VT * T
UNIT = 8 * 8 * 128
PER_TILE = N_TOK // NW
CHUNK = 64
N_CHUNKS = PER_TILE // CHUNK
GROUPS = CHUNK // L


def _lse_body(table_ref, out_ref):
    t = table_ref[...]
    m = jnp.max(t, axis=1, keepdims=True)
    out_ref[...] = m[:, 0] + jnp.log(jnp.sum(jnp.exp(t - m), axis=1))


NU_CEIL = -(-N_UNITS // NW)
assert NU_CEIL % 2 == 0


def _gather_body(idxT_hbm, table_hbm, out_hbm, idxv, slab_v,
                 outbuf_a, outbuf_b, sem_o):
    cid = lax.axis_index("c")
    sid = lax.axis_index("s")
    wid = sid * NC + cid
    ustart = wid * NU_CEIL
    uend = jnp.minimum(ustart + NU_CEIL, N_UNITS)

    pltpu.sync_copy(idxT_hbm, idxv)

    bufs = [outbuf_a, outbuf_b]

    def out_wait(buf):
        pltpu.make_async_copy(buf, out_hbm.at[pl.ds(0, UNIT)], sem_o).wait()

    def unit_body(u, buf):
        vt = u // T
        t = u % T

        @pl.when(jnp.logical_or(t == 0, u == ustart))
        def _():
            pltpu.sync_copy(table_hbm.at[pl.ds(0, VOCAB), pl.ds(vt * 8, 8)],
                            slab_v)

        @pl.when(u - 2 >= ustart)
        def _():
            out_wait(buf)

        def m_body(m, c2):
            for h in range(2):
                mm = m * 2 + h
                bt = mm // 8
                g = mm % 8
                idx16 = idxv[t, pl.ds(bt * 128 + g * 16, L)]
                for vs in range(8):
                    val = plsc.load_gather(
                        slab_v, [idx16, jnp.full((L,), vs, jnp.int32)])
                    buf[pl.ds(bt * 1024 + vs * 128 + g * 16, L)] = val
            return c2

        lax.fori_loop(0, 32, m_body, 0)
        pltpu.async_copy(buf, out_hbm.at[pl.ds((t * VT + vt) * UNIT, UNIT)],
                         sem_o)

    def pair_body(p, carry):
        for h in range(2):
            u = ustart + p * 2 + h

            @pl.when(u < uend)
            def _():
                unit_body(u, bufs[h])
        return carry

    lax.fori_loop(0, NU_CEIL // 2, pair_body, 0)
    out_wait(bufs[0])
    out_wait(bufs[1])


def _loss_body(idx_hbm, tgt_hbm, table1_hbm, lse_hbm, loss_hbm,
               idx_v, tgt_v, fidx_c, tscal_v, lscal_v,
               accv, sums_v, lossv, shared):
    cid = lax.axis_index("c")
    sid = lax.axis_index("s")
    wid = sid * NC + cid
    base = wid * PER_TILE

    pltpu.sync_copy(idx_hbm.at[pl.ds(base, PER_TILE)], idx_v)
    pltpu.sync_copy(tgt_hbm.at[pl.ds(base, PER_TILE)], tgt_v)

    def chunk(c, acc):
        off = c * CHUNK
        for g in range(GROUPS):
            o = off + g * L
            fidx_c[pl.ds(g * L, L)] = idx_v[pl.ds(o, L)] * VOCAB + tgt_v[pl.ds(o, L)]
        pltpu.sync_copy(table1_hbm.at[fidx_c], tscal_v)
        pltpu.sync_copy(lse_hbm.at[idx_v.at[pl.ds(off, CHUNK)]], lscal_v)
        for g in range(GROUPS):
            acc = acc + lscal_v[pl.ds(g * L, L)] - tscal_v[pl.ds(g * L, L)]
        return acc

    acc = lax.fori_loop(0, N_CHUNKS, chunk, jnp.zeros((L,), jnp.float32))
    accv[...] = acc
    pltpu.sync_copy(accv, shared.at[sid])
    plsc.subcore_barrier()

    @pl.when(sid == 0)
    def _():
        pltpu.sync_copy(shared, sums_v)
        tot = sums_v[0]
        for j in range(1, NS):
            tot = tot + sums_v[j]
        lossv[...] = tot * (1.0 / N_TOK)
        pltpu.sync_copy(lossv, loss_hbm.at[cid])


def kernel(idx, targets, table):
    lse = pl.pallas_call(
        _lse_body,
        out_shape=jax.ShapeDtypeStruct((VOCAB,), jnp.float32),
    )(table)

    mesh = plsc.VectorSubcoreMesh(core_axis_name="c", subcore_axis_name="s")
    params = pltpu.CompilerParams(use_tc_tiling_on_sc=False,
                                  needs_layout_passes=False,
                                  disable_bounds_checks=True)

    gather = pl.kernel(
        _gather_body,
        out_type=jax.ShapeDtypeStruct((N_TOK * VOCAB,), jnp.float32),
        mesh=mesh,
        compiler_params=params,
        scratch_types=[
            pltpu.VMEM((T, B), jnp.int32),
            pltpu.VMEM((VOCAB, 8), jnp.float32),
            pltpu.VMEM((UNIT,), jnp.float32),
            pltpu.VMEM((UNIT,), jnp.float32),
            pltpu.SemaphoreType.DMA,
        ],
    )
    flat = gather(jnp.transpose(idx), table)
    o5 = flat.reshape(T, VT, 8, 8, 128)
    logits = o5.transpose(2, 4, 0, 1, 3).reshape(B, T, VOCAB)

    loss_k = pl.kernel(
        _loss_body,
        out_type=jax.ShapeDtypeStruct((NC, L), jnp.float32),
        mesh=mesh,
        compiler_params=params,
        scratch_types=[
            pltpu.VMEM((PER_TILE,), jnp.int32),
            pltpu.VMEM((PER_TILE,), jnp.int32),
            pltpu.VMEM((CHUNK,), jnp.int32),
            pltpu.VMEM((CHUNK,), jnp.float32),
            pltpu.VMEM((CHUNK,), jnp.float32),
            pltpu.VMEM((L,), jnp.float32),
            pltpu.VMEM((NS, L), jnp.float32),
            pltpu.VMEM((L,), jnp.float32),
            pltpu.VMEM_SHARED((NS, L), jnp.float32),
        ],
    )
    loss_parts = loss_k(idx.reshape(N_TOK), targets.reshape(N_TOK),
                        table.reshape(VOCAB * VOCAB), lse)
    loss = jnp.sum(loss_parts)
    return (logits, loss)


# speedup vs baseline: 1.5137x; 1.3955x over previous
import jax
import jax.numpy as jnp
from jax import lax
from jax.experimental import pallas as pl
from jax.experimental.pallas import tpu as pltpu
from jax.experimental.pallas import tpu_sc as plsc

VOCAB = 1000
B, T = 1024, 50
N_TOK = B * T
NC, NS, L = 2, 16, 16
NW = NC * NS
VT = VOCAB // 8
N_UNITS = VT * T
UNIT = 8 * 8 * 128
PER_TILE = N_TOK // NW
CHUNK = 64
N_CHUNKS = PER_TILE // CHUNK
GROUPS = CHUNK // L


def _lse_body(table_ref, out_ref, tt_ref):
    t = table_ref[...]
    m = jnp.max(t, axis=1, keepdims=True)
    out_ref[...] = m[:, 0] + jnp.log(jnp.sum(jnp.exp(t - m), axis=1))
    tt_ref[...] = t.T


NU_CEIL = -(-N_UNITS // NW)
assert NU_CEIL % 2 == 0


def _gather_body(idxT_hbm, tableT_hbm, out_hbm, idxv, slab_v,
                 outbuf_a, outbuf_b, sem_o):
    cid = lax.axis_index("c")
    sid = lax.axis_index("s")
    wid = sid * NC + cid
    ustart = wid * NU_CEIL
    uend = jnp.minimum(ustart + NU_CEIL, N_UNITS)

    pltpu.sync_copy(idxT_hbm, idxv)

    bufs = [outbuf_a, outbuf_b]

    def out_wait(buf):
        pltpu.make_async_copy(buf, out_hbm.at[pl.ds(0, UNIT)], sem_o).wait()

    def unit_body(u, buf):
        vt = u // T
        t = u % T

        @pl.when(jnp.logical_or(t == 0, u == ustart))
        def _():
            pltpu.sync_copy(tableT_hbm.at[pl.ds(vt * 8, 8)], slab_v)

        @pl.when(u - 2 >= ustart)
        def _():
            out_wait(buf)

        def m_body(m, c2):
            for h in range(2):
                mm = m * 2 + h
                bt = mm // 8
                g = mm % 8
                idx16 = idxv[t, pl.ds(bt * 128 + g * 16, L)]
                for vs in range(8):
                    val = plsc.load_gather(
                        slab_v, [jnp.full((L,), vs, jnp.int32), idx16])
                    buf[pl.ds(bt * 1024 + vs * 128 + g * 16, L)] = val
            return c2

        lax.fori_loop(0, 32, m_body, 0)
        pltpu.async_copy(buf, out_hbm.at[pl.ds((t * VT + vt) * UNIT, UNIT)],
                         sem_o)

    def pair_body(p, carry):
        for h in range(2):
            u = ustart + p * 2 + h

            @pl.when(u < uend)
            def _():
                unit_body(u, bufs[h])
        return carry

    lax.fori_loop(0, NU_CEIL // 2, pair_body, 0)
    out_wait(bufs[0])
    out_wait(bufs[1])


def _loss_body(idx_hbm, tgt_hbm, table1_hbm, lse_hbm, loss_hbm,
               idx_v, tgt_v, fidx_c, tscal_v, lscal_v,
               accv, sums_v, lossv, shared):
    cid = lax.axis_index("c")
    sid = lax.axis_index("s")
    wid = sid * NC + cid
    base = wid * PER_TILE

    pltpu.sync_copy(idx_hbm.at[pl.ds(base, PER_TILE)], idx_v)
    pltpu.sync_copy(tgt_hbm.at[pl.ds(base, PER_TILE)], tgt_v)

    def chunk(c, acc):
        off = c * CHUNK
        for g in range(GROUPS):
            o = off + g * L
            fidx_c[pl.ds(g * L, L)] = idx_v[pl.ds(o, L)] * VOCAB + tgt_v[pl.ds(o, L)]
        pltpu.sync_copy(table1_hbm.at[fidx_c], tscal_v)
        pltpu.sync_copy(lse_hbm.at[idx_v.at[pl.ds(off, CHUNK)]], lscal_v)
        for g in range(GROUPS):
            acc = acc + lscal_v[pl.ds(g * L, L)] - tscal_v[pl.ds(g * L, L)]
        return acc

    acc = lax.fori_loop(0, N_CHUNKS, chunk, jnp.zeros((L,), jnp.float32))
    accv[...] = acc
    pltpu.sync_copy(accv, shared.at[sid])
    plsc.subcore_barrier()

    @pl.when(sid == 0)
    def _():
        pltpu.sync_copy(shared, sums_v)
        tot = sums_v[0]
        for j in range(1, NS):
            tot = tot + sums_v[j]
        lossv[...] = tot * (1.0 / N_TOK)
        pltpu.sync_copy(lossv, loss_hbm.at[cid])


def kernel(idx, targets, table):
    lse, table_t = pl.pallas_call(
        _lse_body,
        out_shape=[jax.ShapeDtypeStruct((VOCAB,), jnp.float32),
                   jax.ShapeDtypeStruct((VOCAB, VOCAB), jnp.float32)],
    )(table)

    mesh = plsc.VectorSubcoreMesh(core_axis_name="c", subcore_axis_name="s")
    params = pltpu.CompilerParams(use_tc_tiling_on_sc=False,
                                  needs_layout_passes=False,
                                  disable_bounds_checks=True)

    gather = pl.kernel(
        _gather_body,
        out_type=jax.ShapeDtypeStruct((N_TOK * VOCAB,), jnp.float32),
        mesh=mesh,
        compiler_params=params,
        scratch_types=[
            pltpu.VMEM((T, B), jnp.int32),
            pltpu.VMEM((8, VOCAB), jnp.float32),
            pltpu.VMEM((UNIT,), jnp.float32),
            pltpu.VMEM((UNIT,), jnp.float32),
            pltpu.SemaphoreType.DMA,
        ],
    )
    flat = gather(jnp.transpose(idx), table_t)
    o5 = flat.reshape(T, VT, 8, 8, 128)
    logits = o5.transpose(2, 4, 0, 1, 3).reshape(B, T, VOCAB)

    loss_k = pl.kernel(
        _loss_body,
        out_type=jax.ShapeDtypeStruct((NC, L), jnp.float32),
        mesh=mesh,
        compiler_params=params,
        scratch_types=[
            pltpu.VMEM((PER_TILE,), jnp.int32),
            pltpu.VMEM((PER_TILE,), jnp.int32),
            pltpu.VMEM((CHUNK,), jnp.int32),
            pltpu.VMEM((CHUNK,), jnp.float32),
            pltpu.VMEM((CHUNK,), jnp.float32),
            pltpu.VMEM((L,), jnp.float32),
            pltpu.VMEM((NS, L), jnp.float32),
            pltpu.VMEM((L,), jnp.float32),
            pltpu.VMEM_SHARED((NS, L), jnp.float32),
        ],
    )
    loss_parts = loss_k(idx.reshape(N_TOK), targets.reshape(N_TOK),
                        table.reshape(VOCAB * VOCAB), lse)
    loss = jnp.sum(loss_parts)
    return (logits, loss)
